# baseline (speedup 1.0000x reference)
import functools

import jax
import jax.numpy as jnp
from jax import lax
from jax.experimental import pallas as pl
from jax.experimental.pallas import tpu as pltpu
from jax.experimental.pallas import tpu_sc as plsc

NC = 2
NS = 16
L = 16
NW = NC * NS
ROWS = 128
COLS = 32768
N = ROWS * COLS

SC_ROWS = 48
PER_W = SC_ROWS * COLS // NW
CHUNK = 8192
NCHUNK = PER_W // CHUNK
NBUF = 2
NGRP = NCHUNK // NBUF
U = 8
G = 8
VECS = CHUNK // L

TC_ROW0 = SC_ROWS
BR = 16
QR = 8
NQ = BR // QR
TC_BLK0 = TC_ROW0 // BR
TC_QB0 = TC_ROW0 // QR
NBLK = (ROWS - TC_ROW0) // BR

_INT_MAX = 2**31 - 1

_mesh = plsc.VectorSubcoreMesh(core_axis_name="c", subcore_axis_name="s")


def _chunk_lane_max(bufc):
    init = tuple(jnp.full((L,), -jnp.inf, jnp.float32) for _ in range(U))

    def body(j, accs):
        base = j * (G * L)
        new = list(accs)
        for t in range(G):
            v = bufc[pl.ds(base + t * L, L)]
            k = t % U
            new[k] = jnp.maximum(new[k], v)
        return tuple(new)

    accs = lax.fori_loop(0, VECS // G, body, init)
    m = accs[0]
    for k in range(1, U):
        m = jnp.maximum(m, accs[k])
    return m


@functools.partial(
    pl.kernel,
    out_type=(
        jax.ShapeDtypeStruct((NW, L), jnp.float32),
        jax.ShapeDtypeStruct((NW, L), jnp.int32),
    ),
    mesh=_mesh,
    scratch_types=[
        pltpu.VMEM((NBUF, CHUNK), jnp.float32),
        pltpu.VMEM((L,), jnp.float32),
        pltpu.VMEM((L,), jnp.int32),
    ] + [pltpu.SemaphoreType.DMA] * NBUF,
)
def _scan_kernel(x_hbm, vals_hbm, idxs_hbm, bufs, vout, iout, *sems):
    wid = lax.axis_index("s") * NC + lax.axis_index("c")
    base = wid * PER_W

    def chunk_src_dyn(coff):
        off = base + coff
        return x_hbm.at[off // COLS, pl.ds(off % COLS, CHUNK)]

    for c in range(NBUF):
        pltpu.async_copy(chunk_src_dyn(c * CHUNK), bufs.at[c], sems[c])

    def group(g, carry):
        gm, gc = carry
        for b in range(NBUF):
            idx = g * NBUF + b
            pltpu.make_async_copy(
                chunk_src_dyn(idx * CHUNK), bufs.at[b], sems[b]).wait()
            mc = _chunk_lane_max(bufs.at[b])
            nidx = idx + NBUF

            @pl.when(nidx < NCHUNK)
            def _():
                pltpu.async_copy(chunk_src_dyn(nidx * CHUNK), bufs.at[b], sems[b])

            upd = mc > gm
            gm = jnp.where(upd, mc, gm)
            gc = jnp.where(upd, jnp.full((L,), idx, jnp.int32), gc)
        return gm, gc

    gm, gc = lax.fori_loop(
        0, NGRP, group,
        (jnp.full((L,), -jnp.inf, jnp.float32), jnp.zeros((L,), jnp.int32)))

    gv = gm[0]
    cstar = gc[0]
    for l in range(1, L):
        v = gm[l]
        c = gc[l]
        better = (v > gv) | ((v == gv) & (c < cstar))
        gv = jnp.where(better, v, gv)
        cstar = jnp.where(better, c, cstar)
    gvec = jnp.full((L,), gv, jnp.float32)

    pltpu.async_copy(chunk_src_dyn(cstar * CHUNK), bufs.at[0], sems[0]).wait()

    UB = 2
    iota = lax.iota(jnp.int32, L)
    fis = [jnp.full((L,), _INT_MAX, jnp.int32) for _ in range(UB)]
    idxvs = [iota + k * L for k in range(UB)]

    def locate(j, carry):
        fis, idxvs = carry
        base_j = j * (UB * L)
        nf, ni = [], []
        for k in range(UB):
            v = bufs[0, pl.ds(base_j + k * L, L)]
            cand = jnp.where(v == gvec, idxvs[k], _INT_MAX)
            nf.append(jnp.minimum(fis[k], cand))
            ni.append(idxvs[k] + UB * L)
        return tuple(nf), tuple(ni)

    fis, _ = lax.fori_loop(0, VECS // UB, locate, (tuple(fis), tuple(idxvs)))
    fi = fis[0]
    for k in range(1, UB):
        fi = jnp.minimum(fi, fis[k])
    gi = fi[0]
    for l in range(1, L):
        gi = jnp.minimum(gi, fi[l])
    gi = base + cstar * CHUNK + gi

    vout[...] = gvec
    iout[...] = jnp.full((L,), gi, jnp.int32)
    pltpu.sync_copy(vout, vals_hbm.at[wid])
    pltpu.sync_copy(iout, idxs_hbm.at[wid])


def _tc_pass1_body(x_ref, outv_ref, outb_ref, sm_ref, sb_ref):
    i = pl.program_id(0)
    v = x_ref[...]

    @pl.when(i == 0)
    def _():
        sm_ref[0] = -jnp.inf
        sb_ref[0] = 0

    for q in range(NQ):
        m_q = jnp.max(v[q * QR:(q + 1) * QR, :])
        better = m_q > sm_ref[0]
        sm_ref[0] = jnp.where(better, m_q, sm_ref[0])
        sb_ref[0] = jnp.where(better, TC_QB0 + i * NQ + q, sb_ref[0])

    @pl.when(i == NBLK - 1)
    def _():
        outv_ref[0] = sm_ref[0]
        outb_ref[0] = sb_ref[0]


_tc_pass1 = pl.pallas_call(
    _tc_pass1_body,
    grid=(NBLK,),
    in_specs=[pl.BlockSpec((BR, COLS), lambda i: (TC_BLK0 + i, 0))],
    out_specs=[
        pl.BlockSpec(memory_space=pltpu.SMEM),
        pl.BlockSpec(memory_space=pltpu.SMEM),
    ],
    out_shape=[
        jax.ShapeDtypeStruct((1,), jnp.float32),
        jax.ShapeDtypeStruct((1,), jnp.int32),
    ],
    scratch_shapes=[
        pltpu.SMEM((1,), jnp.float32),
        pltpu.SMEM((1,), jnp.int32),
    ],
)


def _tc_pass2_body(b_ref, x_ref, gv_ref, sc_vals_ref, sc_idxs_ref, out_ref):
    v = x_ref[...]
    m = gv_ref[0]
    ri = lax.broadcasted_iota(jnp.int32, (QR, COLS), 0)
    ci = lax.broadcasted_iota(jnp.int32, (QR, COLS), 1)
    flat = ri * COLS + ci
    local = jnp.min(jnp.where(v == m, flat, _INT_MAX))
    ti = b_ref[0] * (QR * COLS) + local

    sv = sc_vals_ref[...]
    si = sc_idxs_ref[...]
    m_sc = jnp.max(sv)
    i_sc = jnp.min(jnp.where(sv == m_sc, si, _INT_MAX))
    out_ref[0] = jnp.where(m > m_sc, ti, i_sc)


_tc_pass2 = pl.pallas_call(
    _tc_pass2_body,
    grid_spec=pltpu.PrefetchScalarGridSpec(
        num_scalar_prefetch=1,
        grid=(1,),
        in_specs=[
            pl.BlockSpec((QR, COLS), lambda i, b_ref: (b_ref[0], 0)),
            pl.BlockSpec(memory_space=pltpu.SMEM),
            pl.BlockSpec(memory_space=pltpu.VMEM),
            pl.BlockSpec(memory_space=pltpu.VMEM),
        ],
        out_specs=pl.BlockSpec(memory_space=pltpu.SMEM),
    ),
    out_shape=jax.ShapeDtypeStruct((1,), jnp.int32),
)


def kernel(input):
    vals, idxs = _scan_kernel(input)
    tc_v, tc_b = _tc_pass1(input)
    out = _tc_pass2(tc_b, input, tc_v, vals, idxs)
    return out[0].astype(jnp.int64)


# speedup vs baseline: 1.0423x; 1.0423x over previous
import functools

import jax
import jax.numpy as jnp
from jax import lax
from jax.experimental import pallas as pl
from jax.experimental.pallas import tpu as pltpu
from jax.experimental.pallas import tpu_sc as plsc

NC = 2
NS = 16
L = 16
NW = NC * NS
ROWS = 128
COLS = 32768
N = ROWS * COLS

SC_ROWS = 48
PER_W = SC_ROWS * COLS // NW
CHUNK = 8192
NCHUNK = PER_W // CHUNK
NBUF = min(4, NCHUNK)
U = 8
G = 16
VECS = CHUNK // L

TC_ROW0 = SC_ROWS
BR = 16
QR = 8
NQ = BR // QR
TC_BLK0 = TC_ROW0 // BR
TC_QB0 = TC_ROW0 // QR
NBLK = (ROWS - TC_ROW0) // BR

_INT_MAX = 2**31 - 1

_mesh = plsc.VectorSubcoreMesh(core_axis_name="c", subcore_axis_name="s")


def _chunk_lane_max(bufc):
    init = tuple(jnp.full((L,), -jnp.inf, jnp.float32) for _ in range(U))

    def body(j, accs):
        base = j * (G * L)
        new = list(accs)
        for t in range(G):
            v = bufc[pl.ds(base + t * L, L)]
            k = t % U
            new[k] = jnp.maximum(new[k], v)
        return tuple(new)

    accs = lax.fori_loop(0, VECS // G, body, init)
    m = accs[0]
    for k in range(1, U):
        m = jnp.maximum(m, accs[k])
    return m


@functools.partial(
    pl.kernel,
    out_type=(
        jax.ShapeDtypeStruct((NW, L), jnp.float32),
        jax.ShapeDtypeStruct((NW, L), jnp.int32),
    ),
    mesh=_mesh,
    scratch_types=[
        pltpu.VMEM((NBUF, CHUNK), jnp.float32),
        pltpu.VMEM((L,), jnp.float32),
        pltpu.VMEM((L,), jnp.int32),
    ] + [pltpu.SemaphoreType.DMA] * NBUF,
)
def _scan_kernel(x_hbm, vals_hbm, idxs_hbm, bufs, vout, iout, *sems):
    wid = lax.axis_index("s") * NC + lax.axis_index("c")
    base = wid * PER_W

    def chunk_src_dyn(coff):
        off = base + coff
        return x_hbm.at[off // COLS, pl.ds(off % COLS, CHUNK)]

    descs = [None] * NBUF
    for c in range(NBUF - 1):
        descs[c] = pltpu.async_copy(chunk_src_dyn(c * CHUNK), bufs.at[c], sems[c])

    gm = jnp.full((L,), -jnp.inf, jnp.float32)
    gc = jnp.zeros((L,), jnp.int32)
    for c in range(NCHUNK):
        nc_ = c + NBUF - 1
        if nc_ < NCHUNK:
            b = nc_ % NBUF
            descs[b] = pltpu.async_copy(chunk_src_dyn(nc_ * CHUNK), bufs.at[b], sems[b])
        descs[c % NBUF].wait()
        mc = _chunk_lane_max(bufs.at[c % NBUF])
        upd = mc > gm
        gm = jnp.where(upd, mc, gm)
        gc = jnp.where(upd, jnp.full((L,), c, jnp.int32), gc)

    gv = gm[0]
    cstar = gc[0]
    for l in range(1, L):
        v = gm[l]
        c = gc[l]
        better = (v > gv) | ((v == gv) & (c < cstar))
        gv = jnp.where(better, v, gv)
        cstar = jnp.where(better, c, cstar)
    gvec = jnp.full((L,), gv, jnp.float32)

    pltpu.async_copy(chunk_src_dyn(cstar * CHUNK), bufs.at[0], sems[0]).wait()

    UB = 4
    iota = lax.iota(jnp.int32, L)
    fis = [jnp.full((L,), _INT_MAX, jnp.int32) for _ in range(UB)]
    idxvs = [iota + k * L for k in range(UB)]

    def locate(j, carry):
        fis, idxvs = carry
        base_j = j * (UB * L)
        nf, ni = [], []
        for k in range(UB):
            v = bufs[0, pl.ds(base_j + k * L, L)]
            cand = jnp.where(v == gvec, idxvs[k], _INT_MAX)
            nf.append(jnp.minimum(fis[k], cand))
            ni.append(idxvs[k] + UB * L)
        return tuple(nf), tuple(ni)

    fis, _ = lax.fori_loop(0, VECS // UB, locate, (tuple(fis), tuple(idxvs)))
    fi = fis[0]
    for k in range(1, UB):
        fi = jnp.minimum(fi, fis[k])
    gi = fi[0]
    for l in range(1, L):
        gi = jnp.minimum(gi, fi[l])
    gi = base + cstar * CHUNK + gi

    vout[...] = gvec
    iout[...] = jnp.full((L,), gi, jnp.int32)
    pltpu.sync_copy(vout, vals_hbm.at[wid])
    pltpu.sync_copy(iout, idxs_hbm.at[wid])


def _tc_pass1_body(x_ref, outv_ref, outb_ref, sm_ref, sb_ref):
    i = pl.program_id(0)
    v = x_ref[...]

    @pl.when(i == 0)
    def _():
        sm_ref[0] = -jnp.inf
        sb_ref[0] = 0

    for q in range(NQ):
        m_q = jnp.max(v[q * QR:(q + 1) * QR, :])
        better = m_q > sm_ref[0]
        sm_ref[0] = jnp.where(better, m_q, sm_ref[0])
        sb_ref[0] = jnp.where(better, TC_QB0 + i * NQ + q, sb_ref[0])

    @pl.when(i == NBLK - 1)
    def _():
        outv_ref[0] = sm_ref[0]
        outb_ref[0] = sb_ref[0]


_tc_pass1 = pl.pallas_call(
    _tc_pass1_body,
    grid=(NBLK,),
    in_specs=[pl.BlockSpec((BR, COLS), lambda i: (TC_BLK0 + i, 0))],
    out_specs=[
        pl.BlockSpec(memory_space=pltpu.SMEM),
        pl.BlockSpec(memory_space=pltpu.SMEM),
    ],
    out_shape=[
        jax.ShapeDtypeStruct((1,), jnp.float32),
        jax.ShapeDtypeStruct((1,), jnp.int32),
    ],
    scratch_shapes=[
        pltpu.SMEM((1,), jnp.float32),
        pltpu.SMEM((1,), jnp.int32),
    ],
)


def _tc_pass2_body(b_ref, x_ref, gv_ref, sc_vals_ref, sc_idxs_ref, out_ref):
    v = x_ref[...]
    m = gv_ref[0]
    ri = lax.broadcasted_iota(jnp.int32, (QR, COLS), 0)
    ci = lax.broadcasted_iota(jnp.int32, (QR, COLS), 1)
    flat = ri * COLS + ci
    local = jnp.min(jnp.where(v == m, flat, _INT_MAX))
    ti = b_ref[0] * (QR * COLS) + local

    sv = sc_vals_ref[...]
    si = sc_idxs_ref[...]
    m_sc = jnp.max(sv)
    i_sc = jnp.min(jnp.where(sv == m_sc, si, _INT_MAX))
    out_ref[0] = jnp.where(m > m_sc, ti, i_sc)


_tc_pass2 = pl.pallas_call(
    _tc_pass2_body,
    grid_spec=pltpu.PrefetchScalarGridSpec(
        num_scalar_prefetch=1,
        grid=(1,),
        in_specs=[
            pl.BlockSpec((QR, COLS), lambda i, b_ref: (b_ref[0], 0)),
            pl.BlockSpec(memory_space=pltpu.SMEM),
            pl.BlockSpec(memory_space=pltpu.VMEM),
            pl.BlockSpec(memory_space=pltpu.VMEM),
        ],
        out_specs=pl.BlockSpec(memory_space=pltpu.SMEM),
    ),
    out_shape=jax.ShapeDtypeStruct((1,), jnp.int32),
)


def kernel(input):
    vals, idxs = _scan_kernel(input)
    tc_v, tc_b = _tc_pass1(input)
    out = _tc_pass2(tc_b, input, tc_v, vals, idxs)
    return out[0].astype(jnp.int64)


# speedup vs baseline: 1.0727x; 1.0291x over previous
import functools

import jax
import jax.numpy as jnp
from jax import lax
from jax.experimental import pallas as pl
from jax.experimental.pallas import tpu as pltpu
from jax.experimental.pallas import tpu_sc as plsc

NC = 2
NS = 16
L = 16
NW = NC * NS
ROWS = 128
COLS = 32768
N = ROWS * COLS

SC_ROWS = 32
PER_W = SC_ROWS * COLS // NW
CHUNK = 8192
NCHUNK = PER_W // CHUNK
NBUF = min(4, NCHUNK)
U = 8
G = 16
VECS = CHUNK // L

TC_ROW0 = SC_ROWS
BR = 16
QR = 8
NQ = BR // QR
TC_BLK0 = TC_ROW0 // BR
TC_QB0 = TC_ROW0 // QR
NBLK = (ROWS - TC_ROW0) // BR

_INT_MAX = 2**31 - 1

_mesh = plsc.VectorSubcoreMesh(core_axis_name="c", subcore_axis_name="s")


def _chunk_lane_max(bufc):
    init = tuple(jnp.full((L,), -jnp.inf, jnp.float32) for _ in range(U))

    def body(j, accs):
        base = j * (G * L)
        new = list(accs)
        for t in range(G):
            v = bufc[pl.ds(base + t * L, L)]
            k = t % U
            new[k] = jnp.maximum(new[k], v)
        return tuple(new)

    accs = lax.fori_loop(0, VECS // G, body, init)
    m = accs[0]
    for k in range(1, U):
        m = jnp.maximum(m, accs[k])
    return m


@functools.partial(
    pl.kernel,
    out_type=(
        jax.ShapeDtypeStruct((NW, L), jnp.float32),
        jax.ShapeDtypeStruct((NW, L), jnp.int32),
    ),
    mesh=_mesh,
    scratch_types=[
        pltpu.VMEM((NBUF, CHUNK), jnp.float32),
        pltpu.VMEM((L,), jnp.float32),
        pltpu.VMEM((L,), jnp.int32),
    ] + [pltpu.SemaphoreType.DMA] * NBUF,
)
def _scan_kernel(x_hbm, vals_hbm, idxs_hbm, bufs, vout, iout, *sems):
    wid = lax.axis_index("s") * NC + lax.axis_index("c")
    base = wid * PER_W

    def chunk_src_dyn(coff):
        off = base + coff
        return x_hbm.at[off // COLS, pl.ds(off % COLS, CHUNK)]

    descs = [None] * NBUF
    for c in range(NBUF - 1):
        descs[c] = pltpu.async_copy(chunk_src_dyn(c * CHUNK), bufs.at[c], sems[c])

    gm = jnp.full((L,), -jnp.inf, jnp.float32)
    gc = jnp.zeros((L,), jnp.int32)
    for c in range(NCHUNK):
        nc_ = c + NBUF - 1
        if nc_ < NCHUNK:
            b = nc_ % NBUF
            descs[b] = pltpu.async_copy(chunk_src_dyn(nc_ * CHUNK), bufs.at[b], sems[b])
        descs[c % NBUF].wait()
        mc = _chunk_lane_max(bufs.at[c % NBUF])
        upd = mc > gm
        gm = jnp.where(upd, mc, gm)
        gc = jnp.where(upd, jnp.full((L,), c, jnp.int32), gc)

    gv = gm[0]
    cstar = gc[0]
    for l in range(1, L):
        v = gm[l]
        c = gc[l]
        better = (v > gv) | ((v == gv) & (c < cstar))
        gv = jnp.where(better, v, gv)
        cstar = jnp.where(better, c, cstar)
    gvec = jnp.full((L,), gv, jnp.float32)

    pltpu.async_copy(chunk_src_dyn(cstar * CHUNK), bufs.at[0], sems[0]).wait()

    UB = 4
    iota = lax.iota(jnp.int32, L)
    fis = [jnp.full((L,), _INT_MAX, jnp.int32) for _ in range(UB)]
    idxvs = [iota + k * L for k in range(UB)]

    def locate(j, carry):
        fis, idxvs = carry
        base_j = j * (UB * L)
        nf, ni = [], []
        for k in range(UB):
            v = bufs[0, pl.ds(base_j + k * L, L)]
            cand = jnp.where(v == gvec, idxvs[k], _INT_MAX)
            nf.append(jnp.minimum(fis[k], cand))
            ni.append(idxvs[k] + UB * L)
        return tuple(nf), tuple(ni)

    fis, _ = lax.fori_loop(0, VECS // UB, locate, (tuple(fis), tuple(idxvs)))
    fi = fis[0]
    for k in range(1, UB):
        fi = jnp.minimum(fi, fis[k])
    gi = fi[0]
    for l in range(1, L):
        gi = jnp.minimum(gi, fi[l])
    gi = base + cstar * CHUNK + gi

    vout[...] = gvec
    iout[...] = jnp.full((L,), gi, jnp.int32)
    pltpu.sync_copy(vout, vals_hbm.at[wid])
    pltpu.sync_copy(iout, idxs_hbm.at[wid])


def _tc_pass1_body(x_ref, outv_ref, outb_ref, sm_ref, sb_ref):
    i = pl.program_id(0)
    v = x_ref[...]

    @pl.when(i == 0)
    def _():
        sm_ref[0] = -jnp.inf
        sb_ref[0] = 0

    for q in range(NQ):
        m_q = jnp.max(v[q * QR:(q + 1) * QR, :])
        better = m_q > sm_ref[0]
        sm_ref[0] = jnp.where(better, m_q, sm_ref[0])
        sb_ref[0] = jnp.where(better, TC_QB0 + i * NQ + q, sb_ref[0])

    @pl.when(i == NBLK - 1)
    def _():
        outv_ref[0] = sm_ref[0]
        outb_ref[0] = sb_ref[0]


_tc_pass1 = pl.pallas_call(
    _tc_pass1_body,
    grid=(NBLK,),
    in_specs=[pl.BlockSpec((BR, COLS), lambda i: (TC_BLK0 + i, 0))],
    out_specs=[
        pl.BlockSpec(memory_space=pltpu.SMEM),
        pl.BlockSpec(memory_space=pltpu.SMEM),
    ],
    out_shape=[
        jax.ShapeDtypeStruct((1,), jnp.float32),
        jax.ShapeDtypeStruct((1,), jnp.int32),
    ],
    scratch_shapes=[
        pltpu.SMEM((1,), jnp.float32),
        pltpu.SMEM((1,), jnp.int32),
    ],
)


def _tc_pass2_body(b_ref, x_ref, gv_ref, sc_vals_ref, sc_idxs_ref, out_ref):
    v = x_ref[...]
    m = gv_ref[0]
    ri = lax.broadcasted_iota(jnp.int32, (QR, COLS), 0)
    ci = lax.broadcasted_iota(jnp.int32, (QR, COLS), 1)
    flat = ri * COLS + ci
    local = jnp.min(jnp.where(v == m, flat, _INT_MAX))
    ti = b_ref[0] * (QR * COLS) + local

    sv = sc_vals_ref[...]
    si = sc_idxs_ref[...]
    m_sc = jnp.max(sv)
    i_sc = jnp.min(jnp.where(sv == m_sc, si, _INT_MAX))
    out_ref[0] = jnp.where(m > m_sc, ti, i_sc)


_tc_pass2 = pl.pallas_call(
    _tc_pass2_body,
    grid_spec=pltpu.PrefetchScalarGridSpec(
        num_scalar_prefetch=1,
        grid=(1,),
        in_specs=[
            pl.BlockSpec((QR, COLS), lambda i, b_ref: (b_ref[0], 0)),
            pl.BlockSpec(memory_space=pltpu.SMEM),
            pl.BlockSpec(memory_space=pltpu.VMEM),
            pl.BlockSpec(memory_space=pltpu.VMEM),
        ],
        out_specs=pl.BlockSpec(memory_space=pltpu.SMEM),
    ),
    out_shape=jax.ShapeDtypeStruct((1,), jnp.int32),
)


def kernel(input):
    vals, idxs = _scan_kernel(input)
    tc_v, tc_b = _tc_pass1(input)
    out = _tc_pass2(tc_b, input, tc_v, vals, idxs)
    return out[0].astype(jnp.int64)


# speedup vs baseline: 1.0887x; 1.0149x over previous
import functools

import jax
import jax.numpy as jnp
from jax import lax
from jax.experimental import pallas as pl
from jax.experimental.pallas import tpu as pltpu
from jax.experimental.pallas import tpu_sc as plsc

NC = 2
NS = 16
L = 16
NW = NC * NS
ROWS = 128
COLS = 32768
N = ROWS * COLS

SC_ROWS = 16
PER_W = SC_ROWS * COLS // NW
CHUNK = 8192
NCHUNK = PER_W // CHUNK
NBUF = min(4, NCHUNK)
U = 8
G = 16
VECS = CHUNK // L

TC_ROW0 = SC_ROWS
BR = 16
QR = 8
NQ = BR // QR
TC_BLK0 = TC_ROW0 // BR
TC_QB0 = TC_ROW0 // QR
NBLK = (ROWS - TC_ROW0) // BR

_INT_MAX = 2**31 - 1

_mesh = plsc.VectorSubcoreMesh(core_axis_name="c", subcore_axis_name="s")


def _chunk_lane_max(bufc):
    init = tuple(jnp.full((L,), -jnp.inf, jnp.float32) for _ in range(U))

    def body(j, accs):
        base = j * (G * L)
        new = list(accs)
        for t in range(G):
            v = bufc[pl.ds(base + t * L, L)]
            k = t % U
            new[k] = jnp.maximum(new[k], v)
        return tuple(new)

    accs = lax.fori_loop(0, VECS // G, body, init)
    m = accs[0]
    for k in range(1, U):
        m = jnp.maximum(m, accs[k])
    return m


@functools.partial(
    pl.kernel,
    out_type=(
        jax.ShapeDtypeStruct((NW, L), jnp.float32),
        jax.ShapeDtypeStruct((NW, L), jnp.int32),
    ),
    mesh=_mesh,
    scratch_types=[
        pltpu.VMEM((NBUF, CHUNK), jnp.float32),
        pltpu.VMEM((L,), jnp.float32),
        pltpu.VMEM((L,), jnp.int32),
    ] + [pltpu.SemaphoreType.DMA] * NBUF,
)
def _scan_kernel(x_hbm, vals_hbm, idxs_hbm, bufs, vout, iout, *sems):
    wid = lax.axis_index("s") * NC + lax.axis_index("c")
    base = wid * PER_W

    def chunk_src_dyn(coff):
        off = base + coff
        return x_hbm.at[off // COLS, pl.ds(off % COLS, CHUNK)]

    descs = [None] * NBUF
    for c in range(NBUF - 1):
        descs[c] = pltpu.async_copy(chunk_src_dyn(c * CHUNK), bufs.at[c], sems[c])

    gm = jnp.full((L,), -jnp.inf, jnp.float32)
    gc = jnp.zeros((L,), jnp.int32)
    for c in range(NCHUNK):
        nc_ = c + NBUF - 1
        if nc_ < NCHUNK:
            b = nc_ % NBUF
            descs[b] = pltpu.async_copy(chunk_src_dyn(nc_ * CHUNK), bufs.at[b], sems[b])
        descs[c % NBUF].wait()
        mc = _chunk_lane_max(bufs.at[c % NBUF])
        upd = mc > gm
        gm = jnp.where(upd, mc, gm)
        gc = jnp.where(upd, jnp.full((L,), c, jnp.int32), gc)

    gv = gm[0]
    cstar = gc[0]
    for l in range(1, L):
        v = gm[l]
        c = gc[l]
        better = (v > gv) | ((v == gv) & (c < cstar))
        gv = jnp.where(better, v, gv)
        cstar = jnp.where(better, c, cstar)
    gvec = jnp.full((L,), gv, jnp.float32)

    pltpu.async_copy(chunk_src_dyn(cstar * CHUNK), bufs.at[0], sems[0]).wait()

    UB = 4
    iota = lax.iota(jnp.int32, L)
    fis = [jnp.full((L,), _INT_MAX, jnp.int32) for _ in range(UB)]
    idxvs = [iota + k * L for k in range(UB)]

    def locate(j, carry):
        fis, idxvs = carry
        base_j = j * (UB * L)
        nf, ni = [], []
        for k in range(UB):
            v = bufs[0, pl.ds(base_j + k * L, L)]
            cand = jnp.where(v == gvec, idxvs[k], _INT_MAX)
            nf.append(jnp.minimum(fis[k], cand))
            ni.append(idxvs[k] + UB * L)
        return tuple(nf), tuple(ni)

    fis, _ = lax.fori_loop(0, VECS // UB, locate, (tuple(fis), tuple(idxvs)))
    fi = fis[0]
    for k in range(1, UB):
        fi = jnp.minimum(fi, fis[k])
    gi = fi[0]
    for l in range(1, L):
        gi = jnp.minimum(gi, fi[l])
    gi = base + cstar * CHUNK + gi

    vout[...] = gvec
    iout[...] = jnp.full((L,), gi, jnp.int32)
    pltpu.sync_copy(vout, vals_hbm.at[wid])
    pltpu.sync_copy(iout, idxs_hbm.at[wid])


def _tc_pass1_body(x_ref, outv_ref, outb_ref, sm_ref, sb_ref):
    i = pl.program_id(0)
    v = x_ref[...]

    @pl.when(i == 0)
    def _():
        sm_ref[0] = -jnp.inf
        sb_ref[0] = 0

    for q in range(NQ):
        m_q = jnp.max(v[q * QR:(q + 1) * QR, :])
        better = m_q > sm_ref[0]
        sm_ref[0] = jnp.where(better, m_q, sm_ref[0])
        sb_ref[0] = jnp.where(better, TC_QB0 + i * NQ + q, sb_ref[0])

    @pl.when(i == NBLK - 1)
    def _():
        outv_ref[0] = sm_ref[0]
        outb_ref[0] = sb_ref[0]


_tc_pass1 = pl.pallas_call(
    _tc_pass1_body,
    grid=(NBLK,),
    in_specs=[pl.BlockSpec((BR, COLS), lambda i: (TC_BLK0 + i, 0))],
    out_specs=[
        pl.BlockSpec(memory_space=pltpu.SMEM),
        pl.BlockSpec(memory_space=pltpu.SMEM),
    ],
    out_shape=[
        jax.ShapeDtypeStruct((1,), jnp.float32),
        jax.ShapeDtypeStruct((1,), jnp.int32),
    ],
    scratch_shapes=[
        pltpu.SMEM((1,), jnp.float32),
        pltpu.SMEM((1,), jnp.int32),
    ],
)


def _tc_pass2_body(b_ref, x_ref, gv_ref, sc_vals_ref, sc_idxs_ref, out_ref):
    v = x_ref[...]
    m = gv_ref[0]
    ri = lax.broadcasted_iota(jnp.int32, (QR, COLS), 0)
    ci = lax.broadcasted_iota(jnp.int32, (QR, COLS), 1)
    flat = ri * COLS + ci
    local = jnp.min(jnp.where(v == m, flat, _INT_MAX))
    ti = b_ref[0] * (QR * COLS) + local

    sv = sc_vals_ref[...]
    si = sc_idxs_ref[...]
    m_sc = jnp.max(sv)
    i_sc = jnp.min(jnp.where(sv == m_sc, si, _INT_MAX))
    out_ref[0] = jnp.where(m > m_sc, ti, i_sc)


_tc_pass2 = pl.pallas_call(
    _tc_pass2_body,
    grid_spec=pltpu.PrefetchScalarGridSpec(
        num_scalar_prefetch=1,
        grid=(1,),
        in_specs=[
            pl.BlockSpec((QR, COLS), lambda i, b_ref: (b_ref[0], 0)),
            pl.BlockSpec(memory_space=pltpu.SMEM),
            pl.BlockSpec(memory_space=pltpu.VMEM),
            pl.BlockSpec(memory_space=pltpu.VMEM),
        ],
        out_specs=pl.BlockSpec(memory_space=pltpu.SMEM),
    ),
    out_shape=jax.ShapeDtypeStruct((1,), jnp.int32),
)


def kernel(input):
    vals, idxs = _scan_kernel(input)
    tc_v, tc_b = _tc_pass1(input)
    out = _tc_pass2(tc_b, input, tc_v, vals, idxs)
    return out[0].astype(jnp.int64)
